# baseline (speedup 1.0000x reference)
import functools

import jax
import jax.numpy as jnp
from jax import lax
from jax.experimental import pallas as pl
from jax.experimental.pallas import tpu as pltpu
from jax.experimental.pallas import tpu_sc as plsc

NC = 2
NS = 16
NW = NC * NS
L = 16
M = 65536
D = 128
RPW = M // NW
IMAX = 2147483647

BLK = 2048
NBLK = M // BLK

_mesh = plsc.VectorSubcoreMesh(core_axis_name="c", subcore_axis_name="s")
_params = pltpu.CompilerParams(needs_layout_passes=False)


def _mv_body(q_ref, k_ref, s_ref):
    s_ref[...] = jax.lax.dot_general(
        k_ref[...], q_ref[...],
        dimension_numbers=(((1,), (0,)), ((), ())),
        preferred_element_type=jnp.float32)


_mv = pl.pallas_call(
    _mv_body,
    grid=(NBLK,),
    in_specs=[
        pl.BlockSpec((D, 1), lambda b: (0, 0)),
        pl.BlockSpec((BLK, D), lambda b: (b, 0)),
    ],
    out_specs=pl.BlockSpec((BLK, 1), lambda b: (b, 0)),
    out_shape=jax.ShapeDtypeStruct((M, 1), jnp.float32),
)


def _amax_body(s_ref, a_ref, p_ref, bm_ref, bi_ref):
    s = jnp.where(a_ref[...] > 0.0, s_ref[...], -jnp.inf)
    m = jnp.max(s)
    sh = (M // D, D)
    io = (lax.broadcasted_iota(jnp.int32, sh, 0) * D
          + lax.broadcasted_iota(jnp.int32, sh, 1))
    idx = jnp.min(jnp.where(s == m, io, IMAX))
    npc = jnp.where(m > 0.95, idx, p_ref[0, 0])
    io1 = lax.broadcasted_iota(jnp.int32, (1, D), 1)
    bm_ref[...] = jnp.broadcast_to(m, (1, D))
    bi_ref[...] = jnp.where(io1 == 0, idx, jnp.where(io1 == 1, npc, 0))


_amax = pl.pallas_call(
    _amax_body,
    out_shape=[
        jax.ShapeDtypeStruct((1, D), jnp.float32),
        jax.ShapeDtypeStruct((1, D), jnp.int32),
    ],
)


def _finalize_body(bm_hbm, bi_hbm, uc_hbm, av_hbm,
                   ucout_hbm, act_hbm,
                   bmv, biv, ucb, idxv, abuf, sem, semu):
    cid = lax.axis_index("c")
    sid = lax.axis_index("s")
    wid = sid * NC + cid
    wbase = wid * RPW

    cu = pltpu.async_copy(uc_hbm.at[pl.ds(wbase, RPW)], ucb, semu)
    c1 = pltpu.async_copy(bm_hbm, bmv, sem)
    c2 = pltpu.async_copy(bi_hbm, biv, sem)
    c1.wait()
    c2.wait()

    iota = lax.iota(jnp.int32, L)
    mv = bmv[pl.ds(0, 16)]
    iv = biv[pl.ds(0, 16)]
    gm = mv[0]
    gi = iv[0]
    hit = gm > 0.95

    cu.wait()
    off = jnp.clip(gi - wbase, 0, RPW - 1)
    own = (iota == 0) & jnp.broadcast_to(
        hit & (gi >= wbase) & (gi < wbase + RPW), (L,))
    plsc.addupdate_scatter(
        ucb, [jnp.broadcast_to(off, (L,))], jnp.ones((L,), jnp.int32),
        mask=own)
    pltpu.sync_copy(ucb, ucout_hbm.at[pl.ds(wbase, RPW)])

    @pl.when(wid == 0)
    def _():
        idxv[...] = jnp.broadcast_to(gi, (L,))
        pltpu.async_copy(av_hbm.at[idxv], abuf, sem).wait()
        pltpu.sync_copy(abuf.at[0], act_hbm)


_finalize = functools.partial(
    pl.kernel,
    out_type=(
        jax.ShapeDtypeStruct((M,), jnp.int32),
        jax.ShapeDtypeStruct((D,), jnp.float32),
    ),
    mesh=_mesh,
    compiler_params=_params,
    scratch_types=[
        pltpu.VMEM((D,), jnp.float32),
        pltpu.VMEM((D,), jnp.int32),
        pltpu.VMEM((RPW,), jnp.int32),
        pltpu.VMEM((L,), jnp.int32),
        pltpu.VMEM((L, D), jnp.float32),
        pltpu.SemaphoreType.DMA,
        pltpu.SemaphoreType.DMA,
    ],
)(_finalize_body)


def kernel(sensor_spikes, sensor_keys, action_values, is_active,
           usage_counts, program_counter):
    q = sensor_spikes.reshape(D, 1)
    maskf = is_active.astype(jnp.float32).reshape(M // D, D)
    pc11 = program_counter.reshape(1, 1).astype(jnp.int32)

    sim = _mv(q, sensor_keys)
    bm, bi = _amax(sim.reshape(M // D, D), maskf, pc11)
    ucn, act = _finalize(bm.reshape(D), bi.reshape(D),
                         usage_counts, action_values)

    action = act.reshape(1, D)
    confidence = bm[0, 0:1]
    best_idx = bi[0, 0:1]
    new_pc = bi[0, 1]
    return action, confidence, best_idx, ucn, new_pc


# speedup vs baseline: 1.4863x; 1.4863x over previous
import functools

import jax
import jax.numpy as jnp
from jax import lax
from jax.experimental import pallas as pl
from jax.experimental.pallas import tpu as pltpu
from jax.experimental.pallas import tpu_sc as plsc

NC = 2
NS = 16
NW = NC * NS
L = 16
M = 65536
D = 128
RPW = M // NW
IMAX = 2147483647

BLK = 2048
NBLK = M // BLK

_mesh = plsc.VectorSubcoreMesh(core_axis_name="c", subcore_axis_name="s")
_params = pltpu.CompilerParams(needs_layout_passes=False)


def _mv_body(q_ref, k_ref, s_ref):
    s_ref[...] = jax.lax.dot_general(
        q_ref[...], k_ref[...],
        dimension_numbers=(((1,), (1,)), ((), ())),
        preferred_element_type=jnp.float32)


_mv = pl.pallas_call(
    _mv_body,
    grid=(NBLK,),
    in_specs=[
        pl.BlockSpec((1, D), lambda b: (0, 0)),
        pl.BlockSpec((BLK, D), lambda b: (b, 0)),
    ],
    out_specs=pl.BlockSpec((1, BLK), lambda b: (0, b)),
    out_shape=jax.ShapeDtypeStruct((1, M), jnp.float32),
)


def _amax_body(s_ref, a_ref, p_ref, bm_ref, bi_ref):
    s = jnp.where(a_ref[...] > 0.0, s_ref[...], -jnp.inf)
    m = jnp.max(s)
    io = lax.broadcasted_iota(jnp.int32, (1, M), 1)
    idx = jnp.min(jnp.where(s == m, io, IMAX))
    npc = jnp.where(m > 0.95, idx, p_ref[0, 0])
    io1 = lax.broadcasted_iota(jnp.int32, (1, D), 1)
    bm_ref[...] = jnp.broadcast_to(m, (1, D))
    bi_ref[...] = jnp.where(io1 == 0, idx, jnp.where(io1 == 1, npc, 0))


_amax = pl.pallas_call(
    _amax_body,
    out_shape=[
        jax.ShapeDtypeStruct((1, D), jnp.float32),
        jax.ShapeDtypeStruct((1, D), jnp.int32),
    ],
)


def _finalize_body(bm_hbm, bi_hbm, uc_hbm, av_hbm,
                   ucout_hbm, act_hbm,
                   bmv, biv, ucb, idxv, abuf, sem, semu):
    cid = lax.axis_index("c")
    sid = lax.axis_index("s")
    wid = sid * NC + cid
    wbase = wid * RPW

    cu = pltpu.async_copy(uc_hbm.at[pl.ds(wbase, RPW)], ucb, semu)
    c1 = pltpu.async_copy(bm_hbm, bmv, sem)
    c2 = pltpu.async_copy(bi_hbm, biv, sem)
    c1.wait()
    c2.wait()

    iota = lax.iota(jnp.int32, L)
    mv = bmv[pl.ds(0, 16)]
    iv = biv[pl.ds(0, 16)]
    gm = mv[0]
    gi = iv[0]
    hit = gm > 0.95

    cu.wait()
    off = jnp.clip(gi - wbase, 0, RPW - 1)
    own = (iota == 0) & jnp.broadcast_to(
        hit & (gi >= wbase) & (gi < wbase + RPW), (L,))
    plsc.addupdate_scatter(
        ucb, [jnp.broadcast_to(off, (L,))], jnp.ones((L,), jnp.int32),
        mask=own)
    pltpu.sync_copy(ucb, ucout_hbm.at[pl.ds(wbase, RPW)])

    @pl.when(wid == 0)
    def _():
        idxv[...] = jnp.broadcast_to(gi, (L,))
        pltpu.async_copy(av_hbm.at[idxv], abuf, sem).wait()
        pltpu.sync_copy(abuf.at[0], act_hbm)


_finalize = functools.partial(
    pl.kernel,
    out_type=(
        jax.ShapeDtypeStruct((M,), jnp.int32),
        jax.ShapeDtypeStruct((D,), jnp.float32),
    ),
    mesh=_mesh,
    compiler_params=_params,
    scratch_types=[
        pltpu.VMEM((D,), jnp.float32),
        pltpu.VMEM((D,), jnp.int32),
        pltpu.VMEM((RPW,), jnp.int32),
        pltpu.VMEM((L,), jnp.int32),
        pltpu.VMEM((L, D), jnp.float32),
        pltpu.SemaphoreType.DMA,
        pltpu.SemaphoreType.DMA,
    ],
)(_finalize_body)


def kernel(sensor_spikes, sensor_keys, action_values, is_active,
           usage_counts, program_counter):
    maskf = is_active.astype(jnp.float32).reshape(1, M)
    pc11 = program_counter.reshape(1, 1).astype(jnp.int32)

    sim = _mv(sensor_spikes, sensor_keys)
    bm, bi = _amax(sim, maskf, pc11)
    ucn, act = _finalize(bm.reshape(D), bi.reshape(D),
                         usage_counts, action_values)

    action = act.reshape(1, D)
    confidence = bm[0, 0:1]
    best_idx = bi[0, 0:1]
    new_pc = bi[0, 1]
    return action, confidence, best_idx, ucn, new_pc


# speedup vs baseline: 1.7516x; 1.1785x over previous
import functools

import jax
import jax.numpy as jnp
from jax import lax
from jax.experimental import pallas as pl
from jax.experimental.pallas import tpu as pltpu
from jax.experimental.pallas import tpu_sc as plsc

NC = 2
NS = 16
NW = NC * NS
L = 16
M = 65536
D = 128
RPW = M // NW
IMAX = 2147483647

BLK = 4096
NBLK = M // BLK

_mesh = plsc.VectorSubcoreMesh(core_axis_name="c", subcore_axis_name="s")
_params = pltpu.CompilerParams(needs_layout_passes=False,
                               skip_device_barrier=True)


def _mv_body(q_ref, k_ref, s_ref):
    s_ref[...] = jax.lax.dot_general(
        q_ref[...], k_ref[...],
        dimension_numbers=(((1,), (1,)), ((), ())),
        preferred_element_type=jnp.float32)


_mv = pl.pallas_call(
    _mv_body,
    grid=(NBLK,),
    in_specs=[
        pl.BlockSpec((1, D), lambda b: (0, 0)),
        pl.BlockSpec((BLK, D), lambda b: (b, 0)),
    ],
    out_specs=pl.BlockSpec((1, BLK), lambda b: (0, b)),
    out_shape=jax.ShapeDtypeStruct((1, M), jnp.float32),
)


def _amax_body(s_ref, a_ref, p_ref, bm_ref, bi_ref):
    s = jnp.where(a_ref[...], s_ref[...], -jnp.inf)
    m = jnp.max(s)
    io = lax.broadcasted_iota(jnp.int32, (1, M), 1)
    idx = jnp.min(jnp.where(s == m, io, IMAX))
    npc = jnp.where(m > 0.95, idx, p_ref[0, 0])
    io1 = lax.broadcasted_iota(jnp.int32, (1, D), 1)
    bm_ref[...] = jnp.broadcast_to(m, (1, D))
    bi_ref[...] = jnp.where(io1 == 0, idx, jnp.where(io1 == 1, npc, 0))


_amax = pl.pallas_call(
    _amax_body,
    out_shape=[
        jax.ShapeDtypeStruct((1, D), jnp.float32),
        jax.ShapeDtypeStruct((1, D), jnp.int32),
    ],
)


def _finalize_body(bm_hbm, bi_hbm, uc_hbm, av_hbm,
                   ucout_hbm, act_hbm,
                   bmv, biv, ucb, idxv, abuf, sem, semu):
    cid = lax.axis_index("c")
    sid = lax.axis_index("s")
    wid = sid * NC + cid
    wbase = wid * RPW

    cu = pltpu.async_copy(uc_hbm.at[pl.ds(wbase, RPW)], ucb, semu)
    c1 = pltpu.async_copy(bm_hbm, bmv, sem)
    c2 = pltpu.async_copy(bi_hbm, biv, sem)
    c1.wait()
    c2.wait()

    iota = lax.iota(jnp.int32, L)
    mv = bmv[pl.ds(0, 16)]
    iv = biv[pl.ds(0, 16)]
    gm = mv[0]
    gi = iv[0]
    hit = gm > 0.95

    cu.wait()
    off = jnp.clip(gi - wbase, 0, RPW - 1)
    own = (iota == 0) & jnp.broadcast_to(
        hit & (gi >= wbase) & (gi < wbase + RPW), (L,))
    plsc.addupdate_scatter(
        ucb, [jnp.broadcast_to(off, (L,))], jnp.ones((L,), jnp.int32),
        mask=own)
    pltpu.sync_copy(ucb, ucout_hbm.at[pl.ds(wbase, RPW)])

    @pl.when(wid == 0)
    def _():
        idxv[...] = jnp.broadcast_to(gi, (L,))
        pltpu.async_copy(av_hbm.at[idxv], abuf, sem).wait()
        pltpu.sync_copy(abuf.at[0], act_hbm)


_finalize = functools.partial(
    pl.kernel,
    out_type=(
        jax.ShapeDtypeStruct((M,), jnp.int32),
        jax.ShapeDtypeStruct((D,), jnp.float32),
    ),
    mesh=_mesh,
    compiler_params=_params,
    scratch_types=[
        pltpu.VMEM((D,), jnp.float32),
        pltpu.VMEM((D,), jnp.int32),
        pltpu.VMEM((RPW,), jnp.int32),
        pltpu.VMEM((L,), jnp.int32),
        pltpu.VMEM((L, D), jnp.float32),
        pltpu.SemaphoreType.DMA,
        pltpu.SemaphoreType.DMA,
    ],
)(_finalize_body)


def kernel(sensor_spikes, sensor_keys, action_values, is_active,
           usage_counts, program_counter):
    pc11 = program_counter.reshape(1, 1).astype(jnp.int32)

    sim = _mv(sensor_spikes, sensor_keys)
    bm, bi = _amax(sim, is_active.reshape(1, M), pc11)
    ucn, act = _finalize(bm.reshape(D), bi.reshape(D),
                         usage_counts, action_values)

    action = act.reshape(1, D)
    confidence = bm[0, 0:1]
    best_idx = bi[0, 0:1]
    new_pc = bi[0, 1]
    return action, confidence, best_idx, ucn, new_pc
